# baseline (speedup 1.0000x reference)
import functools

import jax
import jax.numpy as jnp
from jax import lax
from jax.experimental import pallas as pl
from jax.experimental.pallas import tpu as pltpu
from jax.experimental.pallas import tpu_sc as plsc

N_NODES = 10000
N_EDGES = 320000
D = 128
CHUNK = 64
GROUP = 32
TOTAL_GROUPS = 160
G0 = 10
G1 = (TOTAL_GROUPS - 16 * G0) // 16
N_WORKERS = 32
E_PAD = TOTAL_GROUPS * GROUP * CHUNK
NBUF = 4
N_PAD = 10240
ROWS_PER_TILE = N_PAD // 16
DROWS = N_PAD // 128

_F32 = jnp.float32


def _make_sc_agg(with_deg: bool):
    scratch = [
        pltpu.VMEM((GROUP, CHUNK), jnp.int32),
        pltpu.VMEM((GROUP, CHUNK), jnp.int32),
    ] + [pltpu.VMEM((CHUNK, D), _F32) for _ in range(NBUF)] + [
        pltpu.VMEM_SHARED((N_PAD, D), _F32),
        pltpu.SemaphoreType.DMA,
        pltpu.SemaphoreType.DMA,
    ]
    out_type = [jax.ShapeDtypeStruct((2, N_PAD, D), _F32)]
    if with_deg:
        out_type.append(jax.ShapeDtypeStruct((2, N_PAD, D), _F32))

    mesh = plsc.VectorSubcoreMesh(core_axis_name="c", subcore_axis_name="s")

    @functools.partial(pl.kernel, mesh=mesh, out_type=tuple(out_type),
                       scratch_types=scratch)
    def k(*refs):
        if with_deg:
            (h_hbm, src_hbm, dst_hbm, zrow_hbm, ones_hbm,
             agg_out, deg_out,
             dsti, srci, rows, rows1, rows2, rows3, acc,
             sem_g, sem_s) = refs
        else:
            (h_hbm, src_hbm, dst_hbm, zrow_hbm,
             agg_out,
             dsti, srci, rows, rows1, rows2, rows3, acc,
             sem_g, sem_s) = refs

        cid = lax.axis_index("c")
        sid = lax.axis_index("s")
        w = cid * 16 + sid
        tstart = sid * ROWS_PER_TILE
        n_slabs = ROWS_PER_TILE // CHUNK

        pltpu.sync_copy(zrow_hbm, rows)
        for r in range(n_slabs):
            pltpu.sync_copy(rows, acc.at[pl.ds(tstart + r * CHUNK, CHUNK)])
        plsc.subcore_barrier()

        bufs = (rows, rows1, rows2, rows3)

        base_g = jnp.where(cid == 0, sid * G0, 16 * G0 + sid * G1)
        n_g = jnp.where(cid == 0, G0, G1)

        def body(g, carry):
            pltpu.sync_copy(dst_hbm.at[base_g + g], dsti)
            pltpu.sync_copy(src_hbm.at[base_g + g], srci)
            gd = [pltpu.async_copy(h_hbm.at[srci.at[b]], bufs[b], sem_g)
                  for b in range(NBUF)]
            sd = [None] * NBUF
            for j in range(GROUP):
                b = j % NBUF
                gd[b].wait()
                sd[b] = pltpu.async_copy(bufs[b], acc.at[dsti.at[j]],
                                         sem_s, add=True)
                if j + NBUF < GROUP:
                    sd[b].wait()
                    sd[b] = None
                    gd[b] = pltpu.async_copy(
                        h_hbm.at[srci.at[j + NBUF]], bufs[b], sem_g)
            for d in sd:
                if d is not None:
                    d.wait()
            return carry

        lax.fori_loop(0, n_g, body, 0)
        plsc.subcore_barrier()

        for r in range(n_slabs):
            o = tstart + r * CHUNK
            pltpu.sync_copy(acc.at[pl.ds(o, CHUNK)], rows)
            pltpu.sync_copy(rows, agg_out.at[cid, pl.ds(o, CHUNK)])

        if with_deg:
            plsc.subcore_barrier()
            pltpu.sync_copy(zrow_hbm, rows)
            for r in range(n_slabs):
                pltpu.sync_copy(rows,
                                acc.at[pl.ds(tstart + r * CHUNK, CHUNK)])
            pltpu.sync_copy(ones_hbm, rows)
            plsc.subcore_barrier()

            def dbody(g, carry):
                pltpu.sync_copy(dst_hbm.at[base_g + g], dsti)
                ds = [pltpu.async_copy(rows, acc.at[dsti.at[j]],
                                       sem_s, add=True)
                      for j in range(GROUP)]
                for d in ds:
                    d.wait()
                return carry

            lax.fori_loop(0, n_g, dbody, 0)
            plsc.subcore_barrier()
            for r in range(n_slabs):
                o = tstart + r * CHUNK
                pltpu.sync_copy(acc.at[pl.ds(o, CHUNK)], rows)
                pltpu.sync_copy(rows, deg_out.at[cid, pl.ds(o, CHUNK)])

    return k


_sc_agg_deg = _make_sc_agg(with_deg=True)
_sc_agg = _make_sc_agg(with_deg=False)

_BR = 1024
_GRID = N_PAD // _BR
_DBR = _BR // 128


def _tc1_body(x_ref, agg_ref, deg_ref, ws_ref, wn_ref, b_ref, o_ref):
    deg = deg_ref[0, :, 0:1] + deg_ref[1, :, 0:1]
    inv = 1.0 / jnp.maximum(deg, 1.0)
    mean = (agg_ref[0] + agg_ref[1]) * inv
    h = jnp.dot(x_ref[...], ws_ref[...], preferred_element_type=_F32)
    h = h + jnp.dot(mean, wn_ref[...], preferred_element_type=_F32)
    o_ref[...] = jnp.maximum(h + b_ref[...], 0.0)


def _tc2_body(h_ref, agg_ref, deg_ref, ws_ref, wn_ref, b2_ref, wc_ref,
              bc_ref, o_ref):
    wsc = jnp.dot(ws_ref[...], wc_ref[...], preferred_element_type=_F32)
    wnc = jnp.dot(wn_ref[...], wc_ref[...], preferred_element_type=_F32)
    bc2 = jnp.dot(b2_ref[...], wc_ref[...], preferred_element_type=_F32) \
        + bc_ref[...]
    deg = deg_ref[0, :, 0:1] + deg_ref[1, :, 0:1]
    inv = 1.0 / jnp.maximum(deg, 1.0)
    mean = (agg_ref[0] + agg_ref[1]) * inv
    o = jnp.dot(h_ref[...], wsc, preferred_element_type=_F32)
    o = o + jnp.dot(mean, wnc, preferred_element_type=_F32)
    o_ref[...] = o + bc2


def _row_spec(width):
    return pl.BlockSpec((_BR, width), lambda i: (i, 0))


def _pair_spec(width):
    return pl.BlockSpec((2, _BR, width), lambda i: (0, i, 0))


def _deg_spec():
    return pl.BlockSpec((2, _BR, 128), lambda i: (0, i, 0))


def _full_spec(r, c):
    return pl.BlockSpec((r, c), lambda i: (0, 0))


_tc1 = pl.pallas_call(
    _tc1_body,
    grid=(_GRID,),
    in_specs=[_row_spec(D), _pair_spec(D), _deg_spec(),
              _full_spec(D, D), _full_spec(D, D), _full_spec(1, D)],
    out_specs=_row_spec(D),
    out_shape=jax.ShapeDtypeStruct((N_PAD, D), _F32),
)

_tc2 = pl.pallas_call(
    _tc2_body,
    grid=(_GRID,),
    in_specs=[_row_spec(D), _pair_spec(D), _deg_spec(),
              _full_spec(D, D), _full_spec(D, D), _full_spec(1, D),
              _full_spec(D, 2), _full_spec(1, 2)],
    out_specs=_row_spec(2),
    out_shape=jax.ShapeDtypeStruct((N_PAD, 2), _F32),
)


def kernel(x, edge_index, W_self1, W_neigh1, b1, W_self2, W_neigh2, b2, Wc,
           bc):
    pad = E_PAD - N_EDGES
    src = jnp.concatenate(
        [edge_index[0].astype(jnp.int32), jnp.zeros((pad,), jnp.int32)]
    ).reshape(TOTAL_GROUPS, GROUP, CHUNK)
    dst = jnp.concatenate(
        [edge_index[1].astype(jnp.int32),
         jnp.full((pad,), N_NODES, jnp.int32)]
    ).reshape(TOTAL_GROUPS, GROUP, CHUNK)
    zrow = jnp.zeros((CHUNK, D), _F32)
    xp = jnp.concatenate([x, jnp.zeros((N_PAD - N_NODES, D), _F32)])

    ones = jnp.ones((CHUNK, D), _F32)
    agg1, deg = _sc_agg_deg(xp, src, dst, zrow, ones)
    h1 = _tc1(xp, agg1, deg, W_self1, W_neigh1, b1.reshape(1, D))
    (agg2,) = _sc_agg(h1, src, dst, zrow)
    out = _tc2(h1, agg2, deg, W_self2, W_neigh2, b2.reshape(1, D), Wc,
               bc.reshape(1, 2))
    return out[:N_NODES]


# speedup vs baseline: 1.3731x; 1.3731x over previous
import functools

import jax
import jax.numpy as jnp
from jax import lax
from jax.experimental import pallas as pl
from jax.experimental.pallas import tpu as pltpu
from jax.experimental.pallas import tpu_sc as plsc

N_NODES = 10000
N_EDGES = 320000
D = 128
CHUNK = 64
GROUP = 32
TOTAL_GROUPS = 160
G0 = 9
G1 = (TOTAL_GROUPS - 16 * G0) // 16
N_WORKERS = 32
E_PAD = TOTAL_GROUPS * GROUP * CHUNK
NBUF = 4
N_PAD = 10240
ROWS_PER_TILE = N_PAD // 16
DROWS = N_PAD // 128

_F32 = jnp.float32


def _make_sc_agg(with_deg: bool):
    scratch = [
        pltpu.VMEM((GROUP, CHUNK), jnp.int32),
        pltpu.VMEM((GROUP, CHUNK), jnp.int32),
    ] + [pltpu.VMEM((CHUNK, D), _F32) for _ in range(NBUF)] + [
        pltpu.VMEM_SHARED((N_PAD, D), _F32),
        pltpu.SemaphoreType.DMA,
        pltpu.SemaphoreType.DMA,
    ]
    out_type = [jax.ShapeDtypeStruct((2, N_PAD, D), _F32)]
    if with_deg:
        out_type.append(jax.ShapeDtypeStruct((2, N_PAD, D), _F32))

    mesh = plsc.VectorSubcoreMesh(core_axis_name="c", subcore_axis_name="s")

    @functools.partial(pl.kernel, mesh=mesh, out_type=tuple(out_type),
                       scratch_types=scratch)
    def k(*refs):
        if with_deg:
            (h_hbm, src_hbm, dst_hbm, zrow_hbm, ones_hbm,
             agg_out, deg_out,
             dsti, srci, rows, rows1, rows2, rows3, acc,
             sem_g, sem_s) = refs
        else:
            (h_hbm, src_hbm, dst_hbm, zrow_hbm,
             agg_out,
             dsti, srci, rows, rows1, rows2, rows3, acc,
             sem_g, sem_s) = refs

        cid = lax.axis_index("c")
        sid = lax.axis_index("s")
        w = cid * 16 + sid
        tstart = sid * ROWS_PER_TILE
        n_slabs = ROWS_PER_TILE // CHUNK

        pltpu.sync_copy(zrow_hbm, rows)
        for r in range(n_slabs):
            pltpu.sync_copy(rows, acc.at[pl.ds(tstart + r * CHUNK, CHUNK)])
        plsc.subcore_barrier()

        bufs = (rows, rows1, rows2, rows3)

        base_g = jnp.where(cid == 0, sid * G0, 16 * G0 + sid * G1)
        n_g = jnp.where(cid == 0, G0, G1)

        def body(g, carry):
            pltpu.sync_copy(dst_hbm.at[base_g + g], dsti)
            pltpu.sync_copy(src_hbm.at[base_g + g], srci)
            gd = [pltpu.async_copy(h_hbm.at[srci.at[b]], bufs[b], sem_g)
                  for b in range(NBUF)]
            sd = [None] * NBUF
            for j in range(GROUP):
                b = j % NBUF
                gd[b].wait()
                sd[b] = pltpu.async_copy(bufs[b], acc.at[dsti.at[j]],
                                         sem_s, add=True)
                if j + NBUF < GROUP:
                    sd[b].wait()
                    sd[b] = None
                    gd[b] = pltpu.async_copy(
                        h_hbm.at[srci.at[j + NBUF]], bufs[b], sem_g)
            for d in sd:
                if d is not None:
                    d.wait()
            return carry

        lax.fori_loop(0, n_g, body, 0)
        plsc.subcore_barrier()

        for r in range(n_slabs):
            o = tstart + r * CHUNK
            pltpu.sync_copy(acc.at[pl.ds(o, CHUNK)], rows)
            pltpu.sync_copy(rows, agg_out.at[cid, pl.ds(o, CHUNK)])

        if with_deg:
            plsc.subcore_barrier()
            pltpu.sync_copy(zrow_hbm, rows)
            for r in range(n_slabs):
                pltpu.sync_copy(rows,
                                acc.at[pl.ds(tstart + r * CHUNK, CHUNK)])
            pltpu.sync_copy(ones_hbm, rows)
            plsc.subcore_barrier()

            def dbody(g, carry):
                pltpu.sync_copy(dst_hbm.at[base_g + g], dsti)
                ds = [pltpu.async_copy(rows, acc.at[dsti.at[j]],
                                       sem_s, add=True)
                      for j in range(GROUP)]
                for d in ds:
                    d.wait()
                return carry

            lax.fori_loop(0, n_g, dbody, 0)
            plsc.subcore_barrier()
            for r in range(n_slabs):
                o = tstart + r * CHUNK
                pltpu.sync_copy(acc.at[pl.ds(o, CHUNK)], rows)
                pltpu.sync_copy(rows, deg_out.at[cid, pl.ds(o, CHUNK)])

    return k


_sc_agg_deg = _make_sc_agg(with_deg=True)
_sc_agg = _make_sc_agg(with_deg=False)

_BR = 1024
_GRID = N_PAD // _BR
_DBR = _BR // 128


def _tc1_body(x_ref, agg_ref, deg_ref, ws_ref, wn_ref, b_ref, o_ref):
    deg = deg_ref[0, :, 0:1] + deg_ref[1, :, 0:1]
    inv = 1.0 / jnp.maximum(deg, 1.0)
    mean = (agg_ref[0] + agg_ref[1]) * inv
    h = jnp.dot(x_ref[...], ws_ref[...], preferred_element_type=_F32)
    h = h + jnp.dot(mean, wn_ref[...], preferred_element_type=_F32)
    o_ref[...] = jnp.maximum(h + b_ref[...], 0.0)


def _tc2_body(h_ref, agg_ref, deg_ref, ws_ref, wn_ref, b2_ref, wc_ref,
              bc_ref, o_ref):
    wsc = jnp.dot(ws_ref[...], wc_ref[...], preferred_element_type=_F32)
    wnc = jnp.dot(wn_ref[...], wc_ref[...], preferred_element_type=_F32)
    bc2 = jnp.dot(b2_ref[...], wc_ref[...], preferred_element_type=_F32) \
        + bc_ref[...]
    deg = deg_ref[0, :, 0:1] + deg_ref[1, :, 0:1]
    inv = 1.0 / jnp.maximum(deg, 1.0)
    mean = (agg_ref[0] + agg_ref[1]) * inv
    o = jnp.dot(h_ref[...], wsc, preferred_element_type=_F32)
    o = o + jnp.dot(mean, wnc, preferred_element_type=_F32)
    o_ref[...] = o + bc2


def _row_spec(width):
    return pl.BlockSpec((_BR, width), lambda i: (i, 0))


def _pair_spec(width):
    return pl.BlockSpec((2, _BR, width), lambda i: (0, i, 0))


def _deg_spec():
    return pl.BlockSpec((2, _BR, 128), lambda i: (0, i, 0))


def _full_spec(r, c):
    return pl.BlockSpec((r, c), lambda i: (0, 0))


_tc1 = pl.pallas_call(
    _tc1_body,
    grid=(_GRID,),
    in_specs=[_row_spec(D), _pair_spec(D), _deg_spec(),
              _full_spec(D, D), _full_spec(D, D), _full_spec(1, D)],
    out_specs=_row_spec(D),
    out_shape=jax.ShapeDtypeStruct((N_PAD, D), _F32),
)

_tc2 = pl.pallas_call(
    _tc2_body,
    grid=(_GRID,),
    in_specs=[_row_spec(D), _pair_spec(D), _deg_spec(),
              _full_spec(D, D), _full_spec(D, D), _full_spec(1, D),
              _full_spec(D, 2), _full_spec(1, 2)],
    out_specs=_row_spec(2),
    out_shape=jax.ShapeDtypeStruct((N_PAD, 2), _F32),
)


def kernel(x, edge_index, W_self1, W_neigh1, b1, W_self2, W_neigh2, b2, Wc,
           bc):
    pad = E_PAD - N_EDGES
    src = jnp.concatenate(
        [edge_index[0].astype(jnp.int32), jnp.zeros((pad,), jnp.int32)]
    ).reshape(TOTAL_GROUPS, GROUP, CHUNK)
    dst = jnp.concatenate(
        [edge_index[1].astype(jnp.int32),
         jnp.full((pad,), N_NODES, jnp.int32)]
    ).reshape(TOTAL_GROUPS, GROUP, CHUNK)
    zrow = jnp.zeros((CHUNK, D), _F32)
    xp = jnp.concatenate([x, jnp.zeros((N_PAD - N_NODES, D), _F32)])

    ones = jnp.ones((CHUNK, D), _F32)
    agg1, deg = _sc_agg_deg(xp, src, dst, zrow, ones)
    h1 = _tc1(xp, agg1, deg, W_self1, W_neigh1, b1.reshape(1, D))
    (agg2,) = _sc_agg(h1, src, dst, zrow)
    out = _tc2(h1, agg2, deg, W_self2, W_neigh2, b2.reshape(1, D), Wc,
               bc.reshape(1, 2))
    return out[:N_NODES]
